# Initial kernel scaffold; baseline (speedup 1.0000x reference)
import jax
import jax.numpy as jnp
from jax.experimental import pallas as pl


def kernel(trigger_edge_index, trigger_edge_weights, x, thrd):
    raise NotImplementedError("write your pallas kernel here")


# speedup vs baseline: 2.1384x; 2.1384x over previous
import functools

import jax
import jax.numpy as jnp
from jax import lax
from jax.experimental import pallas as pl
from jax.experimental.pallas import tpu as pltpu
from jax.experimental.pallas import tpu_sc as plsc

_NC = 2
_NS = 16
_NW = _NC * _NS


def _normalize(x):
    def body(x_ref, o_ref):
        xx = x_ref[...]
        n = jnp.sum(xx * xx, axis=1, keepdims=True)
        o_ref[...] = xx * (1.0 / jnp.maximum(jnp.sqrt(n), 1e-8))

    return pl.pallas_call(
        body,
        out_shape=jax.ShapeDtypeStruct(x.shape, x.dtype),
    )(x)


def _sc_gather(table, idx, chunk):
    B = idx.shape[0]
    D = table.shape[1]
    b_per_w = B // _NW
    mesh = plsc.VectorSubcoreMesh(core_axis_name="c", subcore_axis_name="s")

    @functools.partial(
        pl.kernel,
        mesh=mesh,
        out_type=jax.ShapeDtypeStruct((B, D), table.dtype),
        scratch_types=[
            pltpu.VMEM((chunk,), jnp.int32),
            pltpu.VMEM((chunk, D), table.dtype),
            pltpu.SemaphoreType.DMA,
        ],
    )
    def k(table_hbm, idx_hbm, out_hbm, idx_v, rows_v, sem):
        wid = lax.axis_index("s") * _NC + lax.axis_index("c")
        base = wid * b_per_w

        @pl.loop(0, b_per_w, step=chunk)
        def _(off):
            pltpu.sync_copy(idx_hbm.at[pl.ds(base + off, chunk)], idx_v)
            pltpu.async_copy(table_hbm.at[idx_v], rows_v, sem).wait()
            pltpu.sync_copy(rows_v, out_hbm.at[pl.ds(base + off, chunk)])

    return k(table, idx)


def _masked_cos_loss(g, w, thrd, block):
    E = w.shape[0]
    D = g.shape[1]
    nb = E // block
    w3 = w.reshape(nb, 1, block)
    t = jnp.asarray(thrd, jnp.float32).reshape(1, 1)

    def body(t_ref, s_ref, d_ref, w_ref, o_ref, acc_ref):
        i = pl.program_id(0)

        @pl.when(i == 0)
        def _():
            acc_ref[0] = 0.0
            acc_ref[1] = 0.0

        sims = jnp.sum(s_ref[...] * d_ref[...], axis=1)
        mask = (w_ref[0, 0, :] > 0.0).astype(jnp.float32)
        losses = jnp.maximum(t_ref[0, 0] - sims, 0.0) * mask
        acc_ref[0] += jnp.sum(losses)
        acc_ref[1] += jnp.sum(mask)

        @pl.when(i == nb - 1)
        def _():
            o_ref[0, 0] = acc_ref[0] / jnp.maximum(acc_ref[1], 1.0)

    out = pl.pallas_call(
        body,
        grid=(nb,),
        in_specs=[
            pl.BlockSpec(memory_space=pltpu.SMEM),
            pl.BlockSpec((block, D), lambda i: (i, 0)),
            pl.BlockSpec((block, D), lambda i: (i + nb, 0)),
            pl.BlockSpec((1, 1, block), lambda i: (i, 0, 0)),
        ],
        out_specs=pl.BlockSpec(memory_space=pltpu.SMEM),
        out_shape=jax.ShapeDtypeStruct((1, 1), jnp.float32),
        scratch_shapes=[pltpu.SMEM((2,), jnp.float32)],
    )(t, g, g, w3)
    return out[0, 0]


def kernel(trigger_edge_index, trigger_edge_weights, x, thrd):
    xn = _normalize(x)
    idx = trigger_edge_index.reshape(-1)
    g = _sc_gather(xn, idx, chunk=200)
    return _masked_cos_loss(g, trigger_edge_weights, thrd, block=2000)


# speedup vs baseline: 3.0235x; 1.4139x over previous
import functools

import jax
import jax.numpy as jnp
from jax import lax
from jax.experimental import pallas as pl
from jax.experimental.pallas import tpu as pltpu
from jax.experimental.pallas import tpu_sc as plsc

_NC = 2
_NS = 16
_NW = _NC * _NS


def _normalize(x):
    def body(x_ref, o_ref):
        xx = x_ref[...]
        n = jnp.sum(xx * xx, axis=1, keepdims=True)
        xn = xx * (1.0 / jnp.maximum(jnp.sqrt(n), 1e-8))
        lo = xn[:, :128].astype(jnp.bfloat16).astype(jnp.float32)
        hi = xn[:, 128:].astype(jnp.bfloat16).astype(jnp.float32)
        lo_bits = lax.shift_right_logical(
            lax.bitcast_convert_type(lo, jnp.int32), 16)
        hi_bits = lax.bitcast_convert_type(hi, jnp.int32) & jnp.int32(
            -65536)
        o_ref[...] = lo_bits | hi_bits

    return pl.pallas_call(
        body,
        out_shape=jax.ShapeDtypeStruct((x.shape[0], 128), jnp.int32),
    )(x)


def _sc_gather(table, idx, chunk):
    B = idx.shape[0]
    b_per_w = B // _NW
    mesh = plsc.VectorSubcoreMesh(core_axis_name="c", subcore_axis_name="s")

    @functools.partial(
        pl.kernel,
        mesh=mesh,
        out_type=jax.ShapeDtypeStruct((B,) + table.shape[1:], table.dtype),
        scratch_types=[
            pltpu.VMEM((chunk,), jnp.int32),
            pltpu.VMEM((chunk,) + table.shape[1:], table.dtype),
            pltpu.SemaphoreType.DMA,
        ],
    )
    def k(table_hbm, idx_hbm, out_hbm, idx_v, rows_v, sem):
        wid = lax.axis_index("s") * _NC + lax.axis_index("c")
        base = wid * b_per_w

        @pl.loop(0, b_per_w, step=chunk)
        def _(off):
            pltpu.sync_copy(idx_hbm.at[pl.ds(base + off, chunk)], idx_v)
            pltpu.async_copy(table_hbm.at[idx_v], rows_v, sem).wait()
            pltpu.sync_copy(rows_v, out_hbm.at[pl.ds(base + off, chunk)])

    return k(table, idx)


def _masked_cos_loss(g, w, thrd, block):
    E = w.shape[0]
    nb = E // block
    w3 = w.reshape(nb, 1, block)
    t = jnp.asarray(thrd, jnp.float32).reshape(1, 1)

    def body(t_ref, s_ref, d_ref, w_ref, o_ref, acc_ref):
        i = pl.program_id(0)

        @pl.when(i == 0)
        def _():
            acc_ref[0] = 0.0
            acc_ref[1] = 0.0

        def unpack(v):
            lo = lax.bitcast_convert_type(
                lax.shift_left(v, 16), jnp.float32)
            hi = lax.bitcast_convert_type(v & jnp.int32(-65536),
                                          jnp.float32)
            return lo, hi

        s_lo, s_hi = unpack(s_ref[...])
        d_lo, d_hi = unpack(d_ref[...])
        sims = jnp.sum(s_lo * d_lo + s_hi * d_hi, axis=1)
        mask = (w_ref[0, 0, :] > 0.0).astype(jnp.float32)
        losses = jnp.maximum(t_ref[0, 0] - sims, 0.0) * mask
        acc_ref[0] += jnp.sum(losses)
        acc_ref[1] += jnp.sum(mask)

        @pl.when(i == nb - 1)
        def _():
            o_ref[0, 0] = acc_ref[0] / jnp.maximum(acc_ref[1], 1.0)

    out = pl.pallas_call(
        body,
        grid=(nb,),
        in_specs=[
            pl.BlockSpec(memory_space=pltpu.SMEM),
            pl.BlockSpec((block, 128), lambda i: (i, 0)),
            pl.BlockSpec((block, 128), lambda i: (i + nb, 0)),
            pl.BlockSpec((1, 1, block), lambda i: (i, 0, 0)),
        ],
        out_specs=pl.BlockSpec(memory_space=pltpu.SMEM),
        out_shape=jax.ShapeDtypeStruct((1, 1), jnp.float32),
        scratch_shapes=[pltpu.SMEM((2,), jnp.float32)],
    )(t, g, g, w3)
    return out[0, 0]


def kernel(trigger_edge_index, trigger_edge_weights, x, thrd):
    xn = _normalize(x)
    idx = trigger_edge_index.reshape(-1)
    g = _sc_gather(xn, idx, chunk=200)
    return _masked_cos_loss(g, trigger_edge_weights, thrd, block=2000)


# speedup vs baseline: 3.6561x; 1.2092x over previous
import functools

import jax
import jax.numpy as jnp
from jax import lax
from jax.experimental import pallas as pl
from jax.experimental.pallas import tpu as pltpu
from jax.experimental.pallas import tpu_sc as plsc

_NC = 2
_NS = 16
_NW = _NC * _NS


def _normalize(x):
    def body(x_ref, o_ref):
        xx = x_ref[...]
        n = jnp.sum(xx * xx, axis=1, keepdims=True)
        xn = xx * (1.0 / jnp.maximum(jnp.sqrt(n), 1e-8))
        lo = xn[:, :128].astype(jnp.bfloat16).astype(jnp.float32)
        hi = xn[:, 128:].astype(jnp.bfloat16).astype(jnp.float32)
        lo_bits = lax.shift_right_logical(
            lax.bitcast_convert_type(lo, jnp.int32), 16)
        hi_bits = lax.bitcast_convert_type(hi, jnp.int32) & jnp.int32(
            -65536)
        o_ref[...] = lo_bits | hi_bits

    return pl.pallas_call(
        body,
        out_shape=jax.ShapeDtypeStruct((x.shape[0], 128), jnp.int32),
    )(x)


def _sc_gather(table, idx, chunk):
    B = idx.shape[0]
    b_per_w = B // _NW
    nsteps = b_per_w // chunk
    assert b_per_w % chunk == 0 and nsteps % 2 == 0 and chunk % 8 == 0
    mesh = plsc.VectorSubcoreMesh(core_axis_name="c", subcore_axis_name="s")

    @functools.partial(
        pl.kernel,
        mesh=mesh,
        out_type=jax.ShapeDtypeStruct((B,) + table.shape[1:], table.dtype),
        scratch_types=[
            pltpu.VMEM((chunk,), jnp.int32),
            pltpu.VMEM((chunk,), jnp.int32),
            pltpu.VMEM((chunk,) + table.shape[1:], table.dtype),
            pltpu.VMEM((chunk,) + table.shape[1:], table.dtype),
            pltpu.SemaphoreType.DMA,
            pltpu.SemaphoreType.DMA,
            pltpu.SemaphoreType.DMA,
            pltpu.SemaphoreType.DMA,
        ],
    )
    def k(table_hbm, idx_hbm, out_hbm, idx0, idx1, rows0, rows1,
          g0, g1, o0, o1):
        wid = lax.axis_index("s") * _NC + lax.axis_index("c")
        base = wid * b_per_w
        idxs = (idx0, idx1)
        rows = (rows0, rows1)
        gsem = (g0, g1)
        osem = (o0, o1)

        def start_gather(b, off):
            pltpu.sync_copy(idx_hbm.at[pl.ds(base + off, chunk)], idxs[b])
            pltpu.async_copy(table_hbm.at[idxs[b]], rows[b], gsem[b])

        def wait_gather(b):
            pltpu.make_async_copy(table_hbm.at[idxs[b]], rows[b],
                                  gsem[b]).wait()

        def start_out(b, off):
            pltpu.async_copy(rows[b],
                             out_hbm.at[pl.ds(base + off, chunk)], osem[b])

        def wait_out(b, off):
            pltpu.make_async_copy(rows[b],
                                  out_hbm.at[pl.ds(base + off, chunk)],
                                  osem[b]).wait()

        for b in range(2):
            start_gather(b, b * chunk)

        @pl.loop(0, nsteps - 2, step=2)
        def _(step):
            for b in range(2):
                off = (step + b) * chunk
                wait_gather(b)
                start_out(b, off)
                wait_out(b, off)
                start_gather(b, off + 2 * chunk)

        for b in range(2):
            off = (nsteps - 2 + b) * chunk
            wait_gather(b)
            start_out(b, off)
            wait_out(b, off)

    return k(table, idx)


def _masked_cos_loss(g, w, thrd, block):
    E = w.shape[0]
    nb = E // block
    w3 = w.reshape(nb, 1, block)
    t = jnp.asarray(thrd, jnp.float32).reshape(1, 1)

    def body(t_ref, s_ref, d_ref, w_ref, o_ref, acc_ref):
        i = pl.program_id(0)

        @pl.when(i == 0)
        def _():
            acc_ref[0] = 0.0
            acc_ref[1] = 0.0

        def unpack(v):
            lo = lax.bitcast_convert_type(
                lax.shift_left(v, 16), jnp.float32)
            hi = lax.bitcast_convert_type(v & jnp.int32(-65536),
                                          jnp.float32)
            return lo, hi

        s_lo, s_hi = unpack(s_ref[...])
        d_lo, d_hi = unpack(d_ref[...])
        sims = jnp.sum(s_lo * d_lo + s_hi * d_hi, axis=1)
        mask = (w_ref[0, 0, :] > 0.0).astype(jnp.float32)
        losses = jnp.maximum(t_ref[0, 0] - sims, 0.0) * mask
        acc_ref[0] += jnp.sum(losses)
        acc_ref[1] += jnp.sum(mask)

        @pl.when(i == nb - 1)
        def _():
            o_ref[0, 0] = acc_ref[0] / jnp.maximum(acc_ref[1], 1.0)

    out = pl.pallas_call(
        body,
        grid=(nb,),
        in_specs=[
            pl.BlockSpec(memory_space=pltpu.SMEM),
            pl.BlockSpec((block, 128), lambda i: (i, 0)),
            pl.BlockSpec((block, 128), lambda i: (i + nb, 0)),
            pl.BlockSpec((1, 1, block), lambda i: (i, 0, 0)),
        ],
        out_specs=pl.BlockSpec(memory_space=pltpu.SMEM),
        out_shape=jax.ShapeDtypeStruct((1, 1), jnp.float32),
        scratch_shapes=[pltpu.SMEM((2,), jnp.float32)],
    )(t, g, g, w3)
    return out[0, 0]


def kernel(trigger_edge_index, trigger_edge_weights, x, thrd):
    xn = _normalize(x)
    idx = trigger_edge_index.reshape(-1)
    g = _sc_gather(xn, idx, chunk=200)
    return _masked_cos_loss(g, trigger_edge_weights, thrd, block=2000)


# speedup vs baseline: 4.1346x; 1.1309x over previous
import functools

import jax
import jax.numpy as jnp
from jax import lax
from jax.experimental import pallas as pl
from jax.experimental.pallas import tpu as pltpu
from jax.experimental.pallas import tpu_sc as plsc

_NC = 2
_NS = 16
_NW = _NC * _NS


def _normalize(x):
    def body(x_ref, o_ref):
        xx = x_ref[...]
        n = jnp.sum(xx * xx, axis=1, keepdims=True)
        xn = xx * (1.0 / jnp.maximum(jnp.sqrt(n), 1e-8))
        lo = xn[:, :128].astype(jnp.bfloat16).astype(jnp.float32)
        hi = xn[:, 128:].astype(jnp.bfloat16).astype(jnp.float32)
        lo_bits = lax.shift_right_logical(
            lax.bitcast_convert_type(lo, jnp.int32), 16)
        hi_bits = lax.bitcast_convert_type(hi, jnp.int32) & jnp.int32(
            -65536)
        o_ref[...] = lo_bits | hi_bits

    return pl.pallas_call(
        body,
        out_shape=jax.ShapeDtypeStruct((x.shape[0], 128), jnp.int32),
    )(x)


def _sc_gather(table, idx, chunk):
    B = idx.shape[0]
    b_per_w = B // _NW
    nsteps = b_per_w // chunk
    assert b_per_w % chunk == 0 and nsteps % 2 == 0 and chunk % 8 == 0
    mesh = plsc.VectorSubcoreMesh(core_axis_name="c", subcore_axis_name="s")

    @functools.partial(
        pl.kernel,
        mesh=mesh,
        out_type=jax.ShapeDtypeStruct((B,) + table.shape[1:], table.dtype),
        scratch_types=[
            pltpu.VMEM((chunk,), jnp.int32),
            pltpu.VMEM((chunk,), jnp.int32),
            pltpu.VMEM((chunk,) + table.shape[1:], table.dtype),
            pltpu.VMEM((chunk,) + table.shape[1:], table.dtype),
            pltpu.SemaphoreType.DMA,
            pltpu.SemaphoreType.DMA,
            pltpu.SemaphoreType.DMA,
            pltpu.SemaphoreType.DMA,
        ],
    )
    def k(table_hbm, idx_hbm, out_hbm, idx0, idx1, rows0, rows1,
          g0, g1, o0, o1):
        wid = lax.axis_index("s") * _NC + lax.axis_index("c")
        base = wid * b_per_w
        idxs = (idx0, idx1)
        rows = (rows0, rows1)
        gsem = (g0, g1)
        osem = (o0, o1)

        def start_gather(b, off):
            pltpu.sync_copy(idx_hbm.at[pl.ds(base + off, chunk)], idxs[b])
            pltpu.async_copy(table_hbm.at[idxs[b]], rows[b], gsem[b])

        def wait_gather(b):
            pltpu.make_async_copy(table_hbm.at[idxs[b]], rows[b],
                                  gsem[b]).wait()

        def start_out(b, off):
            pltpu.async_copy(rows[b],
                             out_hbm.at[pl.ds(base + off, chunk)], osem[b])

        def wait_out(b, off):
            pltpu.make_async_copy(rows[b],
                                  out_hbm.at[pl.ds(base + off, chunk)],
                                  osem[b]).wait()

        for b in range(2):
            start_gather(b, b * chunk)

        @pl.loop(0, nsteps - 2, step=2)
        def _(step):
            for b in range(2):
                off = (step + b) * chunk
                wait_gather(b)
                start_out(b, off)
                wait_out(b, off)
                start_gather(b, off + 2 * chunk)

        for b in range(2):
            off = (nsteps - 2 + b) * chunk
            wait_gather(b)
            start_out(b, off)
            wait_out(b, off)

    return k(table, idx)


def _masked_cos_loss(g, w, thrd, block):
    E = w.shape[0]
    nb = E // block
    w3 = w.reshape(nb, 1, block)
    t = jnp.asarray(thrd, jnp.float32).reshape(1, 1)

    def body(t_ref, s_ref, d_ref, w_ref, o_ref, acc_ref):
        i = pl.program_id(0)

        @pl.when(i == 0)
        def _():
            acc_ref[0] = 0.0
            acc_ref[1] = 0.0

        def unpack(v):
            lo = lax.bitcast_convert_type(
                lax.shift_left(v, 16), jnp.float32)
            hi = lax.bitcast_convert_type(v & jnp.int32(-65536),
                                          jnp.float32)
            return lo, hi

        s_lo, s_hi = unpack(s_ref[...])
        d_lo, d_hi = unpack(d_ref[...])
        sims = jnp.sum(s_lo * d_lo + s_hi * d_hi, axis=1)
        mask = (w_ref[0, 0, :] > 0.0).astype(jnp.float32)
        losses = jnp.maximum(t_ref[0, 0] - sims, 0.0) * mask
        acc_ref[0] += jnp.sum(losses)
        acc_ref[1] += jnp.sum(mask)

        @pl.when(i == nb - 1)
        def _():
            o_ref[0, 0] = acc_ref[0] / jnp.maximum(acc_ref[1], 1.0)

    out = pl.pallas_call(
        body,
        grid=(nb,),
        in_specs=[
            pl.BlockSpec(memory_space=pltpu.SMEM),
            pl.BlockSpec((block, 128), lambda i: (i, 0)),
            pl.BlockSpec((block, 128), lambda i: (i + nb, 0)),
            pl.BlockSpec((1, 1, block), lambda i: (i, 0, 0)),
        ],
        out_specs=pl.BlockSpec(memory_space=pltpu.SMEM),
        out_shape=jax.ShapeDtypeStruct((1, 1), jnp.float32),
        scratch_shapes=[pltpu.SMEM((2,), jnp.float32)],
    )(t, g, g, w3)
    return out[0, 0]


def kernel(trigger_edge_index, trigger_edge_weights, x, thrd):
    xn = _normalize(x)
    idx = trigger_edge_index.reshape(-1)
    g = _sc_gather(xn, idx, chunk=200)
    return _masked_cos_loss(g, trigger_edge_weights, thrd, block=6400)


# speedup vs baseline: 4.2905x; 1.0377x over previous
import functools

import jax
import jax.numpy as jnp
from jax import lax
from jax.experimental import pallas as pl
from jax.experimental.pallas import tpu as pltpu
from jax.experimental.pallas import tpu_sc as plsc

_NC = 2
_NS = 16
_NW = _NC * _NS


def _normalize(x):
    def body(x_ref, o_ref):
        xx = x_ref[...]
        n = jnp.sum(xx * xx, axis=1, keepdims=True)
        xn = xx * (1.0 / jnp.maximum(jnp.sqrt(n), 1e-8))
        lo = xn[:, :128].astype(jnp.bfloat16).astype(jnp.float32)
        hi = xn[:, 128:].astype(jnp.bfloat16).astype(jnp.float32)
        lo_bits = lax.shift_right_logical(
            lax.bitcast_convert_type(lo, jnp.int32), 16)
        hi_bits = lax.bitcast_convert_type(hi, jnp.int32) & jnp.int32(
            -65536)
        o_ref[...] = lo_bits | hi_bits

    return pl.pallas_call(
        body,
        out_shape=jax.ShapeDtypeStruct((x.shape[0], 128), jnp.int32),
    )(x)


def _sc_gather(table, idx, chunk):
    B = idx.shape[0]
    b_per_w = B // _NW
    nsteps = b_per_w // chunk
    assert b_per_w % chunk == 0 and nsteps >= 4 and chunk % 8 == 0
    mesh = plsc.VectorSubcoreMesh(core_axis_name="c", subcore_axis_name="s")

    @functools.partial(
        pl.kernel,
        mesh=mesh,
        out_type=jax.ShapeDtypeStruct((B,) + table.shape[1:], table.dtype),
        scratch_types=[
            pltpu.VMEM((chunk,), jnp.int32),
            pltpu.VMEM((chunk,), jnp.int32),
            pltpu.VMEM((chunk,) + table.shape[1:], table.dtype),
            pltpu.VMEM((chunk,) + table.shape[1:], table.dtype),
            pltpu.SemaphoreType.DMA,
            pltpu.SemaphoreType.DMA,
            pltpu.SemaphoreType.DMA,
            pltpu.SemaphoreType.DMA,
        ],
    )
    def k(table_hbm, idx_hbm, out_hbm, idx0, idx1, rows0, rows1,
          g0, g1, o0, o1):
        wid = lax.axis_index("s") * _NC + lax.axis_index("c")
        base = wid * b_per_w
        idxs = (idx0, idx1)
        rows = (rows0, rows1)
        gsem = (g0, g1)
        osem = (o0, o1)

        def start_gather(b, off):
            pltpu.sync_copy(idx_hbm.at[pl.ds(base + off, chunk)], idxs[b])
            pltpu.async_copy(table_hbm.at[idxs[b]], rows[b], gsem[b])

        def wait_gather(b):
            pltpu.make_async_copy(table_hbm.at[idxs[b]], rows[b],
                                  gsem[b]).wait()

        def start_out(b, off):
            pltpu.async_copy(rows[b],
                             out_hbm.at[pl.ds(base + off, chunk)], osem[b])

        def wait_out(b, off):
            pltpu.make_async_copy(rows[b],
                                  out_hbm.at[pl.ds(base + off, chunk)],
                                  osem[b]).wait()

        for b in range(2):
            start_gather(b, b * chunk)

        paired = ((nsteps - 2) // 2) * 2

        @pl.loop(0, paired, step=2)
        def _(step):
            for b in range(2):
                off = (step + b) * chunk
                wait_gather(b)
                start_out(b, off)
                wait_out(b, off)
                start_gather(b, off + 2 * chunk)

        for c in range(paired, nsteps):
            b = c % 2
            off = c * chunk
            wait_gather(b)
            start_out(b, off)
            wait_out(b, off)
            if c + 2 < nsteps:
                start_gather(b, off + 2 * chunk)

    return k(table, idx)


def _partial_loss(g, w, thrd, block):
    E = w.shape[0]
    nb = E // block
    w3 = w.reshape(nb, 1, block)
    t = jnp.asarray(thrd, jnp.float32).reshape(1, 1)

    def body(t_ref, s_ref, d_ref, w_ref, o_ref):
        i = pl.program_id(0)

        @pl.when(i == 0)
        def _():
            o_ref[0] = 0.0
            o_ref[1] = 0.0

        def unpack(v):
            lo = lax.bitcast_convert_type(
                lax.shift_left(v, 16), jnp.float32)
            hi = lax.bitcast_convert_type(v & jnp.int32(-65536),
                                          jnp.float32)
            return lo, hi

        s_lo, s_hi = unpack(s_ref[...])
        d_lo, d_hi = unpack(d_ref[...])
        sims = jnp.sum(s_lo * d_lo + s_hi * d_hi, axis=1)
        mask = (w_ref[0, 0, :] > 0.0).astype(jnp.float32)
        losses = jnp.maximum(t_ref[0, 0] - sims, 0.0) * mask
        o_ref[0] += jnp.sum(losses)
        o_ref[1] += jnp.sum(mask)

    out = pl.pallas_call(
        body,
        grid=(nb,),
        in_specs=[
            pl.BlockSpec(memory_space=pltpu.SMEM),
            pl.BlockSpec((block, 128), lambda i: (i, 0)),
            pl.BlockSpec((block, 128), lambda i: (i + nb, 0)),
            pl.BlockSpec((1, 1, block), lambda i: (i, 0, 0)),
        ],
        out_specs=pl.BlockSpec(memory_space=pltpu.SMEM),
        out_shape=jax.ShapeDtypeStruct((2,), jnp.float32),
    )(t, g, g, w3)
    return out[0], out[1]


def kernel(trigger_edge_index, trigger_edge_weights, x, thrd):
    xn = _normalize(x)
    E = trigger_edge_weights.shape[0]
    h = E // 2
    sums, cnts = [], []
    for k in range(2):
        idx_k = trigger_edge_index[:, k * h:(k + 1) * h].reshape(-1)
        g_k = _sc_gather(xn, idx_k, chunk=200)
        s_k, c_k = _partial_loss(g_k, trigger_edge_weights[k * h:(k + 1) * h],
                                 thrd, block=8000)
        sums.append(s_k)
        cnts.append(c_k)
    return (sums[0] + sums[1]) / jnp.maximum(cnts[0] + cnts[1], 1.0)
